# baseline (speedup 1.0000x reference)
import functools

import jax
import jax.numpy as jnp
from jax import lax
from jax.experimental import pallas as pl
from jax.experimental.pallas import tpu as pltpu
from jax.experimental.pallas import tpu_sc as plsc

N_ROWS = 16384
N_COLS = 1000
LANES = 16
NUM_FULL = N_COLS // LANES
TAIL_OFF = N_COLS - LANES
TAIL_DUP = LANES - (N_COLS - NUM_FULL * LANES)
NC, NS = 2, 16
NW = NC * NS
ROWS_PER_W = N_ROWS // NW
CHUNK = 16
NCHUNK = ROWS_PER_W // CHUNK
NEG_INF = float("-inf")


_GATHER_DNUMS = lax.GatherDimensionNumbers(
    offset_dims=(), collapsed_slice_dims=(0,), start_index_map=(0,))


def _shuf(v, lane, sh):
  return lax.gather(v, (lane ^ sh)[:, None], _GATHER_DNUMS, (1,),
                    mode=lax.GatherScatterMode.PROMISE_IN_BOUNDS)


def _allmax(v, lane):
  for sh in (1, 2, 4, 8):
    v = jnp.maximum(v, _shuf(v, lane, sh))
  return v


def _allmin(v, lane):
  for sh in (1, 2, 4, 8):
    v = jnp.minimum(v, _shuf(v, lane, sh))
  return v


def _allsum(v, lane):
  for sh in (1, 2, 4, 8):
    v = v + _shuf(v, lane, sh)
  return v


def _sc_row_stats(weak, strong):
  mesh = plsc.VectorSubcoreMesh(core_axis_name="c", subcore_axis_name="s")

  @functools.partial(
      pl.kernel,
      mesh=mesh,
      compiler_params=pltpu.CompilerParams(
          use_tc_tiling_on_sc=False, needs_layout_passes=False),
      out_type=(
          jax.ShapeDtypeStruct((N_ROWS,), jnp.float32),
          jax.ShapeDtypeStruct((N_ROWS,), jnp.float32),
      ),
      scratch_types=[
          pltpu.VMEM((CHUNK, N_COLS), jnp.float32),
          pltpu.VMEM((CHUNK, N_COLS), jnp.float32),
          pltpu.VMEM((CHUNK,), jnp.float32),
          pltpu.VMEM((CHUNK,), jnp.float32),
      ],
  )
  def body(weak_hbm, strong_hbm, sumexp_hbm, shift_hbm,
           wbuf, sbuf, s_stage, o_stage):
    wid = lax.axis_index("s") * NC + lax.axis_index("c")
    lane = lax.iota(jnp.int32, LANES)

    def chunk_body(ch, carry):
      row0 = wid * ROWS_PER_W + ch * CHUNK
      pltpu.sync_copy(weak_hbm.at[pl.ds(row0, CHUNK), :], wbuf)
      pltpu.sync_copy(strong_hbm.at[pl.ds(row0, CHUNK), :], sbuf)

      def row_body(r, rcarry):
        svec, mvec, ivec = rcarry

        def wslice(j, mc):
          m, jw = mc
          v = wbuf[r, pl.ds(j * LANES, LANES)]
          take = v > m
          return jnp.maximum(m, v), jnp.where(take, j, jw)

        m_w, j_w = lax.fori_loop(
            0, NUM_FULL, wslice,
            (jnp.full((LANES,), NEG_INF, jnp.float32),
             jnp.zeros((LANES,), jnp.int32)))
        v = wbuf[r, pl.ds(TAIL_OFF, LANES)]
        v = jnp.where(lane >= TAIL_DUP, v, NEG_INF)
        take = v > m_w
        m_w = jnp.maximum(m_w, v)
        j_w = jnp.where(take, NUM_FULL, j_w)
        col = j_w * LANES + lane
        col = jnp.where(j_w == NUM_FULL, col - TAIL_DUP, col)
        mw_max = _allmax(m_w, lane)
        cand = jnp.where(m_w == mw_max, col, jnp.int32(N_COLS))
        target = _allmin(cand, lane)

        def smax(j, m):
          return jnp.maximum(m, sbuf[r, pl.ds(j * LANES, LANES)])

        m_s = lax.fori_loop(0, NUM_FULL, smax,
                            jnp.full((LANES,), NEG_INF, jnp.float32))
        v = sbuf[r, pl.ds(TAIL_OFF, LANES)]
        v = jnp.where(lane >= TAIL_DUP, v, NEG_INF)
        m_s = jnp.maximum(m_s, v)

        def ssum(j, a):
          return a + jnp.exp(sbuf[r, pl.ds(j * LANES, LANES)] - m_s)

        acc = lax.fori_loop(0, NUM_FULL, ssum,
                            jnp.zeros((LANES,), jnp.float32))
        v = sbuf[r, pl.ds(TAIL_OFF, LANES)]
        v = jnp.where(lane >= TAIL_DUP, v, NEG_INF)
        acc = acc + jnp.exp(v - m_s)

        ms_max = _allmax(m_s, lane)
        s_row = _allsum(acc * jnp.exp(m_s - ms_max), lane)

        sel = lane == r
        return (jnp.where(sel, s_row, svec),
                jnp.where(sel, ms_max, mvec),
                jnp.where(sel, target, ivec))

      svec, mvec, ivec = lax.fori_loop(
          0, CHUNK, row_body,
          (jnp.zeros((LANES,), jnp.float32),
           jnp.zeros((LANES,), jnp.float32),
           jnp.zeros((LANES,), jnp.int32)))

      tvec = plsc.load_gather(sbuf, [lane, ivec])
      s_stage[...] = svec
      o_stage[...] = mvec - tvec
      pltpu.sync_copy(s_stage, sumexp_hbm.at[pl.ds(row0, CHUNK)])
      pltpu.sync_copy(o_stage, shift_hbm.at[pl.ds(row0, CHUNK)])
      return carry

    lax.fori_loop(0, NCHUNK, chunk_body, 0)

  return body(weak, strong)


def _tc_finish(sumexp, shift):
  def body(s_ref, o_ref, out_ref):
    out_ref[0, 0] = jnp.sum(o_ref[...] + jnp.log(s_ref[...])) * (1.0 / N_ROWS)

  out = pl.pallas_call(
      body,
      out_shape=jax.ShapeDtypeStruct((1, 1), jnp.float32),
      out_specs=pl.BlockSpec(memory_space=pltpu.SMEM),
  )(sumexp.reshape(128, 128), shift.reshape(128, 128))
  return out[0, 0]


@jax.jit
def _impl(anchors_weak, anchors_strong):
  sumexp, shift = _sc_row_stats(anchors_weak, anchors_strong)
  return _tc_finish(sumexp, shift)


def kernel(head_id, anchors_weak, anchors_strong):
  del head_id
  return _impl(anchors_weak, anchors_strong)


# speedup vs baseline: 2.3610x; 2.3610x over previous
import functools

import jax
import jax.numpy as jnp
from jax import lax
from jax.experimental import pallas as pl
from jax.experimental.pallas import tpu as pltpu
from jax.experimental.pallas import tpu_sc as plsc

N_ROWS = 16384
N_COLS = 1000
LANES = 16
NUM_FULL = N_COLS // LANES
TAIL_OFF = N_COLS - LANES
TAIL_DUP = LANES - (N_COLS - NUM_FULL * LANES)
NC, NS = 2, 16
NW = NC * NS
ROWS_PER_W = N_ROWS // NW
CHUNK = 16
NCHUNK = ROWS_PER_W // CHUNK
SPLIT = NUM_FULL // 2
NEG_INF = float("-inf")

_GATHER_DNUMS = lax.GatherDimensionNumbers(
    offset_dims=(), collapsed_slice_dims=(0,), start_index_map=(0,))


def _shuf(v, lane, sh):
  return lax.gather(v, (lane ^ sh)[:, None], _GATHER_DNUMS, (1,),
                    mode=lax.GatherScatterMode.PROMISE_IN_BOUNDS)


def _allmax(v, lane):
  for sh in (1, 2, 4, 8):
    v = jnp.maximum(v, _shuf(v, lane, sh))
  return v


def _allmin(v, lane):
  for sh in (1, 2, 4, 8):
    v = jnp.minimum(v, _shuf(v, lane, sh))
  return v


def _allsum(v, lane):
  for sh in (1, 2, 4, 8):
    v = v + _shuf(v, lane, sh)
  return v


def _sc_row_stats(weak, strong):
  mesh = plsc.VectorSubcoreMesh(core_axis_name="c", subcore_axis_name="s")

  @functools.partial(
      pl.kernel,
      mesh=mesh,
      compiler_params=pltpu.CompilerParams(
          use_tc_tiling_on_sc=False, needs_layout_passes=False),
      out_type=(
          jax.ShapeDtypeStruct((N_ROWS,), jnp.float32),
          jax.ShapeDtypeStruct((N_ROWS,), jnp.float32),
      ),
      scratch_types=[
          pltpu.VMEM((CHUNK, N_COLS), jnp.float32),
          pltpu.VMEM((CHUNK, N_COLS), jnp.float32),
          pltpu.VMEM((CHUNK, N_COLS), jnp.float32),
          pltpu.VMEM((CHUNK, N_COLS), jnp.float32),
          pltpu.VMEM((ROWS_PER_W,), jnp.float32),
          pltpu.VMEM((ROWS_PER_W,), jnp.float32),
          pltpu.SemaphoreType.DMA,
          pltpu.SemaphoreType.DMA,
          pltpu.SemaphoreType.DMA,
          pltpu.SemaphoreType.DMA,
      ],
  )
  def body(weak_hbm, strong_hbm, sumexp_hbm, shift_hbm,
           wbufA, sbufA, wbufB, sbufB, s_all, o_all,
           semWA, semSA, semWB, semSB):
    wid = lax.axis_index("s") * NC + lax.axis_index("c")
    lane = lax.iota(jnp.int32, LANES)
    row_base = wid * ROWS_PER_W

    def in_slices(ch):
      row0 = row_base + ch * CHUNK
      return (weak_hbm.at[pl.ds(row0, CHUNK), :],
              strong_hbm.at[pl.ds(row0, CHUNK), :])

    def start_chunk(ch, wb, sb, wsem, ssem):
      wsrc, ssrc = in_slices(ch)
      pltpu.async_copy(wsrc, wb, wsem)
      pltpu.async_copy(ssrc, sb, ssem)

    def wait_chunk(ch, wb, sb, wsem, ssem):
      wsrc, ssrc = in_slices(ch)
      pltpu.make_async_copy(wsrc, wb, wsem).wait()
      pltpu.make_async_copy(ssrc, sb, ssem).wait()

    def compute_chunk(ch, wbuf, sbuf):
      def row_body(r, rcarry):
        svec, mvec, ivec = rcarry

        mwA = jnp.full((LANES,), NEG_INF, jnp.float32)
        mwB = mwA
        jwA = jnp.zeros((LANES,), jnp.int32)
        jwB = jwA
        ms = [mwA, mwA, mwA, mwA]
        for j in range(NUM_FULL):
          wv = wbuf[r, pl.ds(j * LANES, LANES)]
          sv = sbuf[r, pl.ds(j * LANES, LANES)]
          if j < SPLIT:
            take = wv > mwA
            mwA = jnp.maximum(mwA, wv)
            jwA = jnp.where(take, j, jwA)
          else:
            take = wv > mwB
            mwB = jnp.maximum(mwB, wv)
            jwB = jnp.where(take, j, jwB)
          ms[j % 4] = jnp.maximum(ms[j % 4], sv)

        takeB = mwB > mwA
        m_w = jnp.maximum(mwA, mwB)
        j_w = jnp.where(takeB, jwB, jwA)
        wv = wbuf[r, pl.ds(TAIL_OFF, LANES)]
        wv = jnp.where(lane >= TAIL_DUP, wv, NEG_INF)
        take = wv > m_w
        m_w = jnp.maximum(m_w, wv)
        j_w = jnp.where(take, NUM_FULL, j_w)
        col = j_w * LANES + lane
        col = jnp.where(j_w == NUM_FULL, col - TAIL_DUP, col)
        mw_max = _allmax(m_w, lane)
        cand = jnp.where(m_w == mw_max, col, jnp.int32(N_COLS))
        target = _allmin(cand, lane)

        m_s = jnp.maximum(jnp.maximum(ms[0], ms[1]),
                          jnp.maximum(ms[2], ms[3]))
        sv = sbuf[r, pl.ds(TAIL_OFF, LANES)]
        sv = jnp.where(lane >= TAIL_DUP, sv, NEG_INF)
        m_s = jnp.maximum(m_s, sv)

        acc = [jnp.zeros((LANES,), jnp.float32) for _ in range(4)]
        for j in range(NUM_FULL):
          sv = sbuf[r, pl.ds(j * LANES, LANES)]
          acc[j % 4] = acc[j % 4] + jnp.exp(sv - m_s)
        sv = sbuf[r, pl.ds(TAIL_OFF, LANES)]
        sv = jnp.where(lane >= TAIL_DUP, sv, NEG_INF)
        a = (acc[0] + acc[1]) + (acc[2] + acc[3]) + jnp.exp(sv - m_s)

        ms_max = _allmax(m_s, lane)
        s_row = _allsum(a * jnp.exp(m_s - ms_max), lane)

        sel = lane == r
        return (jnp.where(sel, s_row, svec),
                jnp.where(sel, ms_max, mvec),
                jnp.where(sel, target, ivec))

      svec, mvec, ivec = lax.fori_loop(
          0, CHUNK, row_body,
          (jnp.zeros((LANES,), jnp.float32),
           jnp.zeros((LANES,), jnp.float32),
           jnp.zeros((LANES,), jnp.int32)))

      tvec = plsc.load_gather(sbuf, [lane, ivec])
      off = ch * CHUNK
      s_all[pl.ds(off, CHUNK)] = svec
      o_all[pl.ds(off, CHUNK)] = mvec - tvec

    start_chunk(0, wbufA, sbufA, semWA, semSA)

    def pair_body(i, carry):
      ch = 2 * i
      start_chunk(ch + 1, wbufB, sbufB, semWB, semSB)
      wait_chunk(ch, wbufA, sbufA, semWA, semSA)
      compute_chunk(ch, wbufA, sbufA)

      @pl.when(ch + 2 < NCHUNK)
      def _():
        start_chunk(ch + 2, wbufA, sbufA, semWA, semSA)

      wait_chunk(ch + 1, wbufB, sbufB, semWB, semSB)
      compute_chunk(ch + 1, wbufB, sbufB)
      return carry

    lax.fori_loop(0, NCHUNK // 2, pair_body, 0)

    pltpu.sync_copy(s_all, sumexp_hbm.at[pl.ds(row_base, ROWS_PER_W)])
    pltpu.sync_copy(o_all, shift_hbm.at[pl.ds(row_base, ROWS_PER_W)])

  return body(weak, strong)


def _tc_finish(sumexp, shift):
  def body(s_ref, o_ref, out_ref):
    out_ref[0, 0] = jnp.sum(o_ref[...] + jnp.log(s_ref[...])) * (1.0 / N_ROWS)

  out = pl.pallas_call(
      body,
      out_shape=jax.ShapeDtypeStruct((1, 1), jnp.float32),
      out_specs=pl.BlockSpec(memory_space=pltpu.SMEM),
  )(sumexp.reshape(128, 128), shift.reshape(128, 128))
  return out[0, 0]


@jax.jit
def _impl(anchors_weak, anchors_strong):
  sumexp, shift = _sc_row_stats(anchors_weak, anchors_strong)
  return _tc_finish(sumexp, shift)


def kernel(head_id, anchors_weak, anchors_strong):
  del head_id
  return _impl(anchors_weak, anchors_strong)


# speedup vs baseline: 2.7285x; 1.1557x over previous
import functools

import jax
import jax.numpy as jnp
from jax import lax
from jax.experimental import pallas as pl
from jax.experimental.pallas import tpu as pltpu
from jax.experimental.pallas import tpu_sc as plsc

N_ROWS = 16384
N_COLS = 1000
LANES = 16
NUM_FULL = N_COLS // LANES
TAIL_OFF = N_COLS - LANES
TAIL_DUP = LANES - (N_COLS - NUM_FULL * LANES)
NC, NS = 2, 16
NW = NC * NS
ROWS_PER_W = N_ROWS // NW
CHUNK = 16
NCHUNK = ROWS_PER_W // CHUNK
SPLIT = NUM_FULL // 2
NEG_INF = float("-inf")

_GATHER_DNUMS = lax.GatherDimensionNumbers(
    offset_dims=(), collapsed_slice_dims=(0,), start_index_map=(0,))


def _shuf(v, lane, sh):
  return lax.gather(v, (lane ^ sh)[:, None], _GATHER_DNUMS, (1,),
                    mode=lax.GatherScatterMode.PROMISE_IN_BOUNDS)


def _allmax(v, lane):
  for sh in (1, 2, 4, 8):
    v = jnp.maximum(v, _shuf(v, lane, sh))
  return v


def _allmin(v, lane):
  for sh in (1, 2, 4, 8):
    v = jnp.minimum(v, _shuf(v, lane, sh))
  return v


def _allsum(v, lane):
  for sh in (1, 2, 4, 8):
    v = v + _shuf(v, lane, sh)
  return v


def _sc_row_stats(weak, strong):
  mesh = plsc.VectorSubcoreMesh(core_axis_name="c", subcore_axis_name="s")

  @functools.partial(
      pl.kernel,
      mesh=mesh,
      compiler_params=pltpu.CompilerParams(
          use_tc_tiling_on_sc=True, needs_layout_passes=False),
      out_type=(
          jax.ShapeDtypeStruct((N_ROWS,), jnp.float32),
          jax.ShapeDtypeStruct((N_ROWS,), jnp.float32),
      ),
      scratch_types=[
          pltpu.VMEM((CHUNK, N_COLS), jnp.float32),
          pltpu.VMEM((CHUNK, N_COLS), jnp.float32),
          pltpu.VMEM((CHUNK, N_COLS), jnp.float32),
          pltpu.VMEM((CHUNK, N_COLS), jnp.float32),
          pltpu.VMEM((ROWS_PER_W,), jnp.float32),
          pltpu.VMEM((ROWS_PER_W,), jnp.float32),
          pltpu.SemaphoreType.DMA,
          pltpu.SemaphoreType.DMA,
          pltpu.SemaphoreType.DMA,
          pltpu.SemaphoreType.DMA,
      ],
  )
  def body(weak_hbm, strong_hbm, sumexp_hbm, shift_hbm,
           wbufA, sbufA, wbufB, sbufB, s_all, o_all,
           semWA, semSA, semWB, semSB):
    wid = lax.axis_index("s") * NC + lax.axis_index("c")
    lane = lax.iota(jnp.int32, LANES)
    row_base = wid * ROWS_PER_W

    def in_slices(ch):
      row0 = row_base + ch * CHUNK
      return (weak_hbm.at[pl.ds(row0, CHUNK), :],
              strong_hbm.at[pl.ds(row0, CHUNK), :])

    def start_chunk(ch, wb, sb, wsem, ssem):
      wsrc, ssrc = in_slices(ch)
      pltpu.async_copy(wsrc, wb, wsem)
      pltpu.async_copy(ssrc, sb, ssem)

    def wait_chunk(ch, wb, sb, wsem, ssem):
      wsrc, ssrc = in_slices(ch)
      pltpu.make_async_copy(wsrc, wb, wsem).wait()
      pltpu.make_async_copy(ssrc, sb, ssem).wait()

    def compute_chunk(ch, wbuf, sbuf):
      def row_body(r, rcarry):
        svec, mvec, ivec = rcarry

        mwA = jnp.full((LANES,), NEG_INF, jnp.float32)
        mwB = mwA
        jwA = jnp.zeros((LANES,), jnp.int32)
        jwB = jwA
        ms = [mwA, mwA, mwA, mwA]
        for j in range(NUM_FULL):
          wv = wbuf[r, pl.ds(j * LANES, LANES)]
          sv = sbuf[r, pl.ds(j * LANES, LANES)]
          if j < SPLIT:
            take = wv > mwA
            mwA = jnp.maximum(mwA, wv)
            jwA = jnp.where(take, j, jwA)
          else:
            take = wv > mwB
            mwB = jnp.maximum(mwB, wv)
            jwB = jnp.where(take, j, jwB)
          ms[j % 4] = jnp.maximum(ms[j % 4], sv)

        takeB = mwB > mwA
        m_w = jnp.maximum(mwA, mwB)
        j_w = jnp.where(takeB, jwB, jwA)
        wv = wbuf[r, pl.ds(TAIL_OFF, LANES)]
        wv = jnp.where(lane >= TAIL_DUP, wv, NEG_INF)
        take = wv > m_w
        m_w = jnp.maximum(m_w, wv)
        j_w = jnp.where(take, NUM_FULL, j_w)
        col = j_w * LANES + lane
        col = jnp.where(j_w == NUM_FULL, col - TAIL_DUP, col)
        mw_max = _allmax(m_w, lane)
        cand = jnp.where(m_w == mw_max, col, jnp.int32(N_COLS))
        target = _allmin(cand, lane)

        m_s = jnp.maximum(jnp.maximum(ms[0], ms[1]),
                          jnp.maximum(ms[2], ms[3]))
        sv = sbuf[r, pl.ds(TAIL_OFF, LANES)]
        sv = jnp.where(lane >= TAIL_DUP, sv, NEG_INF)
        m_s = jnp.maximum(m_s, sv)

        acc = [jnp.zeros((LANES,), jnp.float32) for _ in range(4)]
        for j in range(NUM_FULL):
          sv = sbuf[r, pl.ds(j * LANES, LANES)]
          acc[j % 4] = acc[j % 4] + jnp.exp(sv - m_s)
        sv = sbuf[r, pl.ds(TAIL_OFF, LANES)]
        sv = jnp.where(lane >= TAIL_DUP, sv, NEG_INF)
        a = (acc[0] + acc[1]) + (acc[2] + acc[3]) + jnp.exp(sv - m_s)

        ms_max = _allmax(m_s, lane)
        s_row = _allsum(a * jnp.exp(m_s - ms_max), lane)

        sel = lane == r
        return (jnp.where(sel, s_row, svec),
                jnp.where(sel, ms_max, mvec),
                jnp.where(sel, target, ivec))

      svec, mvec, ivec = lax.fori_loop(
          0, CHUNK, row_body,
          (jnp.zeros((LANES,), jnp.float32),
           jnp.zeros((LANES,), jnp.float32),
           jnp.zeros((LANES,), jnp.int32)))

      tvec = plsc.load_gather(sbuf, [lane, ivec])
      off = ch * CHUNK
      s_all[pl.ds(off, CHUNK)] = svec
      o_all[pl.ds(off, CHUNK)] = mvec - tvec

    start_chunk(0, wbufA, sbufA, semWA, semSA)

    def pair_body(i, carry):
      ch = 2 * i
      start_chunk(ch + 1, wbufB, sbufB, semWB, semSB)
      wait_chunk(ch, wbufA, sbufA, semWA, semSA)
      compute_chunk(ch, wbufA, sbufA)

      @pl.when(ch + 2 < NCHUNK)
      def _():
        start_chunk(ch + 2, wbufA, sbufA, semWA, semSA)

      wait_chunk(ch + 1, wbufB, sbufB, semWB, semSB)
      compute_chunk(ch + 1, wbufB, sbufB)
      return carry

    lax.fori_loop(0, NCHUNK // 2, pair_body, 0)

    pltpu.sync_copy(s_all, sumexp_hbm.at[pl.ds(row_base, ROWS_PER_W)])
    pltpu.sync_copy(o_all, shift_hbm.at[pl.ds(row_base, ROWS_PER_W)])

  return body(weak, strong)


def _tc_finish(sumexp, shift):
  def body(s_ref, o_ref, out_ref):
    out_ref[0, 0] = jnp.sum(o_ref[...] + jnp.log(s_ref[...])) * (1.0 / N_ROWS)

  out = pl.pallas_call(
      body,
      out_shape=jax.ShapeDtypeStruct((1, 1), jnp.float32),
      out_specs=pl.BlockSpec(memory_space=pltpu.SMEM),
  )(sumexp.reshape(128, 128), shift.reshape(128, 128))
  return out[0, 0]


@jax.jit
def _impl(anchors_weak, anchors_strong):
  sumexp, shift = _sc_row_stats(anchors_weak, anchors_strong)
  return _tc_finish(sumexp, shift)


def kernel(head_id, anchors_weak, anchors_strong):
  del head_id
  return _impl(anchors_weak, anchors_strong)


# speedup vs baseline: 3.5992x; 1.3191x over previous
import functools

import jax
import jax.numpy as jnp
from jax import lax
from jax.experimental import pallas as pl
from jax.experimental.pallas import tpu as pltpu
from jax.experimental.pallas import tpu_sc as plsc

N_ROWS = 16384
N_COLS = 1000
LANES = 16
NUM_FULL = N_COLS // LANES
TAIL_OFF = N_COLS - LANES
TAIL_DUP = LANES - (N_COLS - NUM_FULL * LANES)
NC, NS = 2, 16
NW = NC * NS
ROWS_PER_W = N_ROWS // NW
CHUNK = 16
NCHUNK = ROWS_PER_W // CHUNK
SPLIT = NUM_FULL // 2
NEG_INF = float("-inf")

_GATHER_DNUMS = lax.GatherDimensionNumbers(
    offset_dims=(), collapsed_slice_dims=(0,), start_index_map=(0,))


def _shuf(v, lane, sh):
  return lax.gather(v, (lane ^ sh)[:, None], _GATHER_DNUMS, (1,),
                    mode=lax.GatherScatterMode.PROMISE_IN_BOUNDS)


def _allmax(v, lane):
  for sh in (1, 2, 4, 8):
    v = jnp.maximum(v, _shuf(v, lane, sh))
  return v


def _allmin(v, lane):
  for sh in (1, 2, 4, 8):
    v = jnp.minimum(v, _shuf(v, lane, sh))
  return v


def _tree(vals, op):
  vals = list(vals)
  while len(vals) > 1:
    nxt = [op(vals[i], vals[i + 1]) for i in range(0, len(vals) - 1, 2)]
    if len(vals) % 2:
      nxt.append(vals[-1])
    vals = nxt
  return vals[0]


def _sc_row_stats(weak, strong):
  mesh = plsc.VectorSubcoreMesh(core_axis_name="c", subcore_axis_name="s")

  @functools.partial(
      pl.kernel,
      mesh=mesh,
      compiler_params=pltpu.CompilerParams(
          use_tc_tiling_on_sc=True, needs_layout_passes=False),
      out_type=(
          jax.ShapeDtypeStruct((N_ROWS // 4, 128), jnp.float32),
          jax.ShapeDtypeStruct((N_ROWS,), jnp.float32),
      ),
      scratch_types=[
          pltpu.VMEM((CHUNK, N_COLS), jnp.float32),
          pltpu.VMEM((CHUNK, N_COLS), jnp.float32),
          pltpu.VMEM((CHUNK, N_COLS), jnp.float32),
          pltpu.VMEM((CHUNK, N_COLS), jnp.float32),
          pltpu.VMEM((ROWS_PER_W // 4, 128), jnp.float32),
          pltpu.VMEM((ROWS_PER_W,), jnp.float32),
          pltpu.SemaphoreType.DMA,
          pltpu.SemaphoreType.DMA,
          pltpu.SemaphoreType.DMA,
          pltpu.SemaphoreType.DMA,
      ],
  )
  def body(weak_hbm, strong_hbm, stats_hbm, tval_hbm,
           wbufA, sbufA, wbufB, sbufB, st_all, t_all,
           semWA, semSA, semWB, semSB):
    wid = lax.axis_index("s") * NC + lax.axis_index("c")
    lane = lax.iota(jnp.int32, LANES)
    row_base = wid * ROWS_PER_W

    def in_slices(ch):
      row0 = row_base + ch * CHUNK
      return (weak_hbm.at[pl.ds(row0, CHUNK), :],
              strong_hbm.at[pl.ds(row0, CHUNK), :])

    def start_chunk(ch, wb, sb, wsem, ssem):
      wsrc, ssrc = in_slices(ch)
      pltpu.async_copy(wsrc, wb, wsem)
      pltpu.async_copy(ssrc, sb, ssem)

    def wait_chunk(ch, wb, sb, wsem, ssem):
      wsrc, ssrc = in_slices(ch)
      pltpu.make_async_copy(wsrc, wb, wsem).wait()
      pltpu.make_async_copy(ssrc, sb, ssem).wait()

    def compute_chunk(ch, wbuf, sbuf):
      def row_body(r, ivec):
        mwA = jnp.full((LANES,), NEG_INF, jnp.float32)
        mwB = mwA
        jwA = jnp.zeros((LANES,), jnp.int32)
        jwB = jwA
        for j in range(NUM_FULL):
          wv = wbuf[r, pl.ds(j * LANES, LANES)]
          if j < SPLIT:
            take = wv > mwA
            mwA = jnp.maximum(mwA, wv)
            jwA = jnp.where(take, j, jwA)
          else:
            take = wv > mwB
            mwB = jnp.maximum(mwB, wv)
            jwB = jnp.where(take, j, jwB)
        takeB = mwB > mwA
        m_w = jnp.maximum(mwA, mwB)
        j_w = jnp.where(takeB, jwB, jwA)
        wv = wbuf[r, pl.ds(TAIL_OFF, LANES)]
        wv = jnp.where(lane >= TAIL_DUP, wv, NEG_INF)
        take = wv > m_w
        m_w = jnp.maximum(m_w, wv)
        j_w = jnp.where(take, NUM_FULL, j_w)
        col = j_w * LANES + lane
        col = jnp.where(j_w == NUM_FULL, col - TAIL_DUP, col)
        mw_max = _allmax(m_w, lane)
        cand = jnp.where(m_w == mw_max, col, jnp.int32(N_COLS))
        target = _allmin(cand, lane)

        pairs = []
        for b0, b1 in ((0, 16), (16, 32), (32, 48), (48, 63)):
          vs = [sbuf[r, pl.ds(j * LANES, LANES)]
                for j in range(b0, min(b1, NUM_FULL))]
          if b1 > NUM_FULL:
            sv = sbuf[r, pl.ds(TAIL_OFF, LANES)]
            vs.append(jnp.where(lane >= TAIL_DUP, sv, NEG_INF))
          mb = _tree(vs, jnp.maximum)
          ab = _tree([jnp.exp(v - mb) for v in vs], jnp.add)
          pairs.append((mb, ab))

        def comb(p, q):
          (m1, a1), (m2, a2) = p, q
          m = jnp.maximum(m1, m2)
          return m, a1 * jnp.exp(m1 - m) + a2 * jnp.exp(m2 - m)

        m_s, a_s = comb(comb(pairs[0], pairs[1]), comb(pairs[2], pairs[3]))

        idx = ch * CHUNK + r
        prow = idx // 4
        pcol = (idx % 4) * 2 * LANES
        st_all[prow, pl.ds(pcol, LANES)] = m_s
        st_all[prow, pl.ds(pcol + LANES, LANES)] = a_s
        return jnp.where(lane == r, target, ivec)

      ivec = lax.fori_loop(0, CHUNK, row_body,
                           jnp.zeros((LANES,), jnp.int32))
      tvec = plsc.load_gather(sbuf, [lane, ivec])
      t_all[pl.ds(ch * CHUNK, CHUNK)] = tvec

    start_chunk(0, wbufA, sbufA, semWA, semSA)

    def pair_body(i, carry):
      ch = 2 * i
      start_chunk(ch + 1, wbufB, sbufB, semWB, semSB)
      wait_chunk(ch, wbufA, sbufA, semWA, semSA)
      compute_chunk(ch, wbufA, sbufA)

      @pl.when(ch + 2 < NCHUNK)
      def _():
        start_chunk(ch + 2, wbufA, sbufA, semWA, semSA)

      wait_chunk(ch + 1, wbufB, sbufB, semWB, semSB)
      compute_chunk(ch + 1, wbufB, sbufB)
      return carry

    lax.fori_loop(0, NCHUNK // 2, pair_body, 0)

    stat_base = wid * (ROWS_PER_W // 4)
    pltpu.sync_copy(
        st_all, stats_hbm.at[pl.ds(stat_base, ROWS_PER_W // 4), :])
    pltpu.sync_copy(t_all, tval_hbm.at[pl.ds(row_base, ROWS_PER_W)])

  return body(weak, strong)


def _tc_finish(stats, tvals):
  def body(st_ref, t_ref, out_ref):
    acc = jnp.float32(0.0)
    for g in range(4):
      m = st_ref[:, g * 2 * LANES:g * 2 * LANES + LANES]
      a = st_ref[:, g * 2 * LANES + LANES:(g + 1) * 2 * LANES]
      rmax = jnp.max(m, axis=1, keepdims=True)
      lse = rmax[:, 0] + jnp.log(jnp.sum(a * jnp.exp(m - rmax), axis=1))
      acc = acc + jnp.sum(lse)
    out_ref[0, 0] = (acc - jnp.sum(t_ref[...])) * (1.0 / N_ROWS)

  out = pl.pallas_call(
      body,
      out_shape=jax.ShapeDtypeStruct((1, 1), jnp.float32),
      out_specs=pl.BlockSpec(memory_space=pltpu.SMEM),
  )(stats, tvals.reshape(128, 128))
  return out[0, 0]


@jax.jit
def _impl(anchors_weak, anchors_strong):
  stats, tvals = _sc_row_stats(anchors_weak, anchors_strong)
  return _tc_finish(stats, tvals)


def kernel(head_id, anchors_weak, anchors_strong):
  del head_id
  return _impl(anchors_weak, anchors_strong)
